# baseline (speedup 1.0000x reference)
import functools

import jax
import jax.numpy as jnp
from jax import lax
from jax.experimental import pallas as pl
from jax.experimental.pallas import tpu as pltpu
from jax.experimental.pallas import tpu_sc as plsc

N = 10000
E = 320000
H = 128
G = 128

NC = 2
NS = 16
LANES = 16

EK = 128
EP = 327680
TILE_E = EP // (NC * NS)
NCHUNK = TILE_E // EK
NSLOT = 2
GRP = 8
NGRP = NCHUNK // GRP
NP = 10240
ROWS_T = NP // NS
ZR = 128

_mesh = plsc.VectorSubcoreMesh(
    core_axis_name="c", subcore_axis_name="s", num_cores=NC, num_subcores=NS
)


@functools.partial(
    pl.kernel,
    out_type=jax.ShapeDtypeStruct((NC, NP, H), jnp.float32),
    mesh=_mesh,
    scratch_types=[
        pltpu.VMEM((NCHUNK, EK), jnp.int32),
        pltpu.VMEM((EK, H), jnp.float32),
        pltpu.VMEM((ZR, H), jnp.float32),
        pltpu.VMEM_SHARED((NP, H), jnp.float32),
        pltpu.SemaphoreType.DMA,
    ],
)
def _sc_degree(dst_hbm, out_hbm, didx_all, ones_v, zbuf, acc, sem):
    c = lax.axis_index("c")
    s = lax.axis_index("s")

    def fill(k, _):
        ones_v[k // (H // LANES), pl.ds((k % (H // LANES)) * LANES, LANES)] = (
            jnp.full((LANES,), 1.0, jnp.float32)
        )
        return 0

    lax.fori_loop(0, EK * (H // LANES), fill, 0)

    def fillz(k, _):
        zbuf[k // (H // LANES), pl.ds((k % (H // LANES)) * LANES, LANES)] = (
            jnp.zeros((LANES,), jnp.float32)
        )
        return 0

    lax.fori_loop(0, ZR * (H // LANES), fillz, 0)

    def zero_acc(j, _):
        pltpu.sync_copy(zbuf, acc.at[pl.ds(s * ROWS_T + j * ZR, ZR)])
        return 0

    lax.fori_loop(0, ROWS_T // ZR, zero_acc, 0)

    wid = c * NS + s
    pltpu.sync_copy(dst_hbm.at[pl.ds(wid * NCHUNK, NCHUNK)], didx_all)
    plsc.subcore_barrier()

    def fire(i, _):
        pltpu.async_copy(ones_v, acc.at[didx_all.at[i]], sem, add=True)
        return 0

    lax.fori_loop(0, NCHUNK, fire, 0)

    def drain(i, _):
        pltpu.make_async_copy(ones_v, acc.at[didx_all.at[0]], sem).wait()
        return 0

    lax.fori_loop(0, NCHUNK, drain, 0)
    plsc.subcore_barrier()

    pltpu.sync_copy(
        acc.at[pl.ds(s * ROWS_T, ROWS_T)],
        out_hbm.at[c, pl.ds(s * ROWS_T, ROWS_T)],
    )


@functools.partial(
    pl.kernel,
    out_type=jax.ShapeDtypeStruct((NC, NP, H), jnp.float32),
    mesh=_mesh,
    scratch_types=[
        [pltpu.VMEM((GRP, EK), jnp.int32)] * 2,
        [pltpu.VMEM((GRP, EK), jnp.int32)] * 2,
        [pltpu.VMEM((EK, H), jnp.float32)] * NSLOT,
        pltpu.VMEM_SHARED((NP, H), jnp.float32),
        [pltpu.SemaphoreType.DMA] * NSLOT,
        [pltpu.SemaphoreType.DMA] * 2,
    ],
)
def _sc_agg(h_hbm, src_hbm, dst_hbm, out_hbm, sidx, didx, rows, acc,
            gsem, isem):
    c = lax.axis_index("c")
    s = lax.axis_index("s")

    def fillz(k, _):
        rows[0][k // (H // LANES), pl.ds((k % (H // LANES)) * LANES, LANES)] = (
            jnp.zeros((LANES,), jnp.float32)
        )
        return 0

    lax.fori_loop(0, EK * (H // LANES), fillz, 0)

    def zero_acc(j, _):
        pltpu.sync_copy(rows[0], acc.at[pl.ds(s * ROWS_T + j * ZR, ZR)])
        return 0

    lax.fori_loop(0, ROWS_T // ZR, zero_acc, 0)
    wid = c * NS + s
    plsc.subcore_barrier()

    def load_idx_async(g, p):
        base = wid * NCHUNK + g * GRP
        pltpu.async_copy(src_hbm.at[pl.ds(base, GRP)], sidx[p], isem[p])
        pltpu.async_copy(dst_hbm.at[pl.ds(base, GRP)], didx[p], isem[p])

    def wait_idx(p):
        pltpu.make_async_copy(src_hbm.at[pl.ds(0, GRP)], sidx[p], isem[p]).wait()
        pltpu.make_async_copy(dst_hbm.at[pl.ds(0, GRP)], didx[p], isem[p]).wait()

    def fire_gather(idx_row, t):
        pltpu.async_copy(h_hbm.at[idx_row], rows[t], gsem[t])

    def process_group(p, cross):
        for j in range(GRP):
            t = j % NSLOT
            if j < GRP - 1:
                fire_gather(sidx[p].at[j + 1], (j + 1) % NSLOT)
            elif cross:
                fire_gather(sidx[1 - p].at[0], 0)
            pltpu.make_async_copy(h_hbm.at[sidx[p].at[j]], rows[t], gsem[t]).wait()
            pltpu.sync_copy(rows[t], acc.at[didx[p].at[j]], add=True)

    pltpu.sync_copy(src_hbm.at[pl.ds(wid * NCHUNK, GRP)], sidx[0])
    pltpu.sync_copy(dst_hbm.at[pl.ds(wid * NCHUNK, GRP)], didx[0])
    load_idx_async(1, 1)
    fire_gather(sidx[0].at[0], 0)

    def super_body(sg, _):
        ga = 2 * sg

        @pl.when(sg > 0)
        def _():
            wait_idx(0)
            fire_gather(sidx[0].at[0], 0)

        wait_idx(1)
        process_group(0, cross=True)

        @pl.when(sg < NGRP // 2 - 1)
        def _():
            load_idx_async(ga + 2, 0)

        process_group(1, cross=False)

        @pl.when(sg < NGRP // 2 - 1)
        def _():
            load_idx_async(ga + 3, 1)

        return 0

    lax.fori_loop(0, NGRP // 2, super_body, 0)
    plsc.subcore_barrier()

    pltpu.sync_copy(
        acc.at[pl.ds(s * ROWS_T, ROWS_T)],
        out_hbm.at[c, pl.ds(s * ROWS_T, ROWS_T)],
    )


BM = 1000
NB = N // BM


def _tc_mm1_body(x_ref, w_ref, degp_ref, h_ref, dinv_ref):
    dinv = lax.rsqrt(degp_ref[0] + degp_ref[1] + 1.0)
    h = jnp.dot(x_ref[...], w_ref[...], preferred_element_type=jnp.float32)
    dinv_ref[...] = dinv
    h_ref[...] = h * dinv


def _bn_phase0(aggp_ref, hp_ref, dinv_ref, b_ref, zs_ref, stat_ref, i):
    @pl.when(i == 0)
    def _():
        stat_ref[...] = jnp.zeros((8, H), jnp.float32)

    z = (aggp_ref[0] + aggp_ref[1] + hp_ref[...]) * dinv_ref[...] + b_ref[...]
    zs_ref[pl.ds(i * BM, BM), :] = z
    stat_ref[0:1, :] += jnp.sum(z, axis=0, keepdims=True)
    stat_ref[1:2, :] += jnp.sum(z * z, axis=0, keepdims=True)


def _bn_phase1_y(g_ref, be_ref, zs_ref, stat_ref, i):
    @pl.when(i == 0)
    def _():
        mu = stat_ref[0:1, :] * (1.0 / N)
        var = stat_ref[1:2, :] * (1.0 / N) - mu * mu
        stat_ref[2:3, :] = mu
        stat_ref[3:4, :] = lax.rsqrt(var + 1e-5)

    z = zs_ref[pl.ds(i * BM, BM), :]
    return jnp.maximum(
        (z - stat_ref[2:3, :]) * stat_ref[3:4, :] * g_ref[...] + be_ref[...], 0.0
    )


def _tc_bn_mm_body(aggp_ref, hp_ref, dinv_ref, b_ref, g_ref, be_ref, w_ref,
                   o_ref, zs_ref, stat_ref):
    p = pl.program_id(0)
    i = pl.program_id(1)

    @pl.when(p == 0)
    def _():
        _bn_phase0(aggp_ref, hp_ref, dinv_ref, b_ref, zs_ref, stat_ref, i)
        o_ref[...] = jnp.zeros((BM, H), jnp.float32)

    @pl.when(p == 1)
    def _():
        y = _bn_phase1_y(g_ref, be_ref, zs_ref, stat_ref, i)
        o_ref[...] = (
            jnp.dot(y, w_ref[...], preferred_element_type=jnp.float32)
            * dinv_ref[...]
        )


def _tc_bn_pool_body(aggp_ref, hp_ref, dinv_ref, b_ref, g_ref, be_ref,
                     batch_ref, fc1w_ref, fc1b_ref, fc2w_ref, fc2b_ref, o_ref,
                     zs_ref, stat_ref, pool_ref, cnt_ref):
    p = pl.program_id(0)
    i = pl.program_id(1)

    @pl.when(p == 0)
    def _():
        _bn_phase0(aggp_ref, hp_ref, dinv_ref, b_ref, zs_ref, stat_ref, i)
        o_ref[...] = jnp.zeros((G, H), jnp.float32)

    @pl.when(p == 1)
    def _():
        @pl.when(i == 0)
        def _():
            pool_ref[...] = jnp.zeros((G, H), jnp.float32)
            cnt_ref[...] = jnp.zeros((G, H), jnp.float32)

        y = _bn_phase1_y(g_ref, be_ref, zs_ref, stat_ref, i)
        gid = lax.broadcasted_iota(jnp.int32, (G, BM), 0)
        onehot_t = (batch_ref[0] == gid).astype(jnp.float32)
        pool_ref[...] += jnp.dot(
            onehot_t, y, preferred_element_type=jnp.float32
        )
        cnt_ref[...] += jnp.broadcast_to(
            jnp.sum(onehot_t, axis=1, keepdims=True), (G, H)
        )

    @pl.when((p == 2) & (i == 0))
    def _():
        pooled = pool_ref[...] / jnp.maximum(cnt_ref[...], 1.0)
        a = jnp.maximum(
            jnp.dot(pooled, fc1w_ref[...], preferred_element_type=jnp.float32)
            + fc1b_ref[...],
            0.0,
        )
        o_ref[...] = (
            jnp.dot(a, fc2w_ref[...], preferred_element_type=jnp.float32)
            + fc2b_ref[...]
        )


def kernel(x, edge_index, batch, W1, b1, gamma1, beta1, W2, b2, gamma2, beta2,
           fc1_W, fc1_b, fc2_W, fc2_b):
    pad_iota = jnp.arange(EP - E, dtype=jnp.int32)
    src = jnp.concatenate([edge_index[0], pad_iota % N]).reshape(EP // EK, EK)
    dst = jnp.concatenate([edge_index[1], N + pad_iota % (NP - N)]).reshape(
        EP // EK, EK
    )

    degp = _sc_degree(dst)

    row_spec = pl.BlockSpec((BM, H), lambda i: (i, 0))
    h1p, dinv = pl.pallas_call(
        _tc_mm1_body,
        grid=(NB,),
        in_specs=[
            row_spec,
            pl.BlockSpec((H, H), lambda i: (0, 0)),
            pl.BlockSpec((2, BM, H), lambda i: (0, i, 0)),
        ],
        out_specs=[row_spec, row_spec],
        out_shape=[
            jax.ShapeDtypeStruct((N, H), jnp.float32),
            jax.ShapeDtypeStruct((N, H), jnp.float32),
        ],
    )(x, W1, degp)

    agg1 = _sc_agg(h1p, src, dst)

    def p0_map3(p, i):
        return (0, jnp.where(p == 0, i, 0), 0)

    def p0_map2(p, i):
        return (jnp.where(p == 0, i, 0), 0)

    prow_spec = pl.BlockSpec((BM, H), lambda p, i: (i, 0))
    wide_spec = pl.BlockSpec((1, H), lambda p, i: (0, 0))
    bn_scratch = [
        pltpu.VMEM((N, H), jnp.float32),
        pltpu.VMEM((8, H), jnp.float32),
    ]

    h2p = pl.pallas_call(
        _tc_bn_mm_body,
        grid=(2, NB),
        in_specs=[
            pl.BlockSpec((2, BM, H), p0_map3),
            pl.BlockSpec((BM, H), p0_map2),
            prow_spec,
            wide_spec, wide_spec, wide_spec,
            pl.BlockSpec((H, H), lambda p, i: (0, 0)),
        ],
        out_specs=prow_spec,
        out_shape=jax.ShapeDtypeStruct((N, H), jnp.float32),
        scratch_shapes=bn_scratch,
    )(agg1, h1p, dinv, b1.reshape(1, H), gamma1.reshape(1, H),
      beta1.reshape(1, H), W2)

    agg2 = _sc_agg(h2p, src, dst)

    fc1w_p = jnp.pad(fc1_W, ((0, 0), (0, 128 - fc1_W.shape[1])))
    fc1b_p = jnp.pad(fc1_b, (0, 128 - fc1_b.shape[0])).reshape(1, 128)
    fc2w_p = jnp.pad(fc2_W, ((0, 128 - fc2_W.shape[0]), (0, 128 - fc2_W.shape[1])))
    fc2b_p = jnp.pad(fc2_b, (0, 128 - fc2_b.shape[0])).reshape(1, 128)

    out_p = pl.pallas_call(
        _tc_bn_pool_body,
        grid=(3, NB),
        in_specs=[
            pl.BlockSpec((2, BM, H), p0_map3),
            pl.BlockSpec((BM, H), p0_map2),
            prow_spec,
            wide_spec, wide_spec, wide_spec,
            pl.BlockSpec((1, 1, BM), lambda p, i: (i, 0, 0)),
            pl.BlockSpec((H, 128), lambda p, i: (0, 0)),
            pl.BlockSpec((1, 128), lambda p, i: (0, 0)),
            pl.BlockSpec((128, 128), lambda p, i: (0, 0)),
            pl.BlockSpec((1, 128), lambda p, i: (0, 0)),
        ],
        out_specs=pl.BlockSpec((G, 128), lambda p, i: (0, 0)),
        out_shape=jax.ShapeDtypeStruct((G, 128), jnp.float32),
        scratch_shapes=bn_scratch + [
            pltpu.VMEM((G, H), jnp.float32),
            pltpu.VMEM((G, H), jnp.float32),
        ],
    )(agg2, h2p, dinv, b2.reshape(1, H), gamma2.reshape(1, H),
      beta2.reshape(1, H), batch.reshape(NB, 1, BM), fc1w_p, fc1b_p, fc2w_p,
      fc2b_p)

    return out_p[:, : fc2_W.shape[1]]


# speedup vs baseline: 1.0601x; 1.0601x over previous
import functools

import jax
import jax.numpy as jnp
from jax import lax
from jax.experimental import pallas as pl
from jax.experimental.pallas import tpu as pltpu
from jax.experimental.pallas import tpu_sc as plsc

N = 10000
E = 320000
H = 128
G = 128

NC = 2
NS = 16
LANES = 16

EK = 128
EP = 327680
TILE_E = EP // (NC * NS)
NCHUNK = TILE_E // EK
NSLOT = 2
GRP = 8
NGRP = NCHUNK // GRP
NP = 10240
ROWS_T = NP // NS
ZR = 128

_mesh = plsc.VectorSubcoreMesh(
    core_axis_name="c", subcore_axis_name="s", num_cores=NC, num_subcores=NS
)


@functools.partial(
    pl.kernel,
    out_type=jax.ShapeDtypeStruct((NC, NP, H), jnp.float32),
    mesh=_mesh,
    scratch_types=[
        pltpu.VMEM((NCHUNK, EK), jnp.int32),
        pltpu.VMEM((EK, H), jnp.float32),
        pltpu.VMEM((ZR, H), jnp.float32),
        pltpu.VMEM_SHARED((NP, H), jnp.float32),
        pltpu.SemaphoreType.DMA,
    ],
)
def _sc_degree(dst_hbm, out_hbm, didx_all, ones_v, zbuf, acc, sem):
    c = lax.axis_index("c")
    s = lax.axis_index("s")

    def fill(k, _):
        ones_v[k // (H // LANES), pl.ds((k % (H // LANES)) * LANES, LANES)] = (
            jnp.full((LANES,), 1.0, jnp.float32)
        )
        return 0

    lax.fori_loop(0, EK * (H // LANES), fill, 0)

    def fillz(k, _):
        zbuf[k // (H // LANES), pl.ds((k % (H // LANES)) * LANES, LANES)] = (
            jnp.zeros((LANES,), jnp.float32)
        )
        return 0

    lax.fori_loop(0, ZR * (H // LANES), fillz, 0)

    def zero_acc(j, _):
        pltpu.sync_copy(zbuf, acc.at[pl.ds(s * ROWS_T + j * ZR, ZR)])
        return 0

    lax.fori_loop(0, ROWS_T // ZR, zero_acc, 0)

    wid = c * NS + s
    pltpu.sync_copy(dst_hbm.at[pl.ds(wid * NCHUNK, NCHUNK)], didx_all)
    plsc.subcore_barrier()

    def fire(i, _):
        pltpu.async_copy(ones_v, acc.at[didx_all.at[i]], sem, add=True)
        return 0

    lax.fori_loop(0, NCHUNK, fire, 0)

    def drain(i, _):
        pltpu.make_async_copy(ones_v, acc.at[didx_all.at[0]], sem).wait()
        return 0

    lax.fori_loop(0, NCHUNK, drain, 0)
    plsc.subcore_barrier()

    pltpu.sync_copy(
        acc.at[pl.ds(s * ROWS_T, ROWS_T)],
        out_hbm.at[c, pl.ds(s * ROWS_T, ROWS_T)],
    )


@functools.partial(
    pl.kernel,
    out_type=jax.ShapeDtypeStruct((NC, NP, H), jnp.float32),
    mesh=_mesh,
    scratch_types=[
        [pltpu.VMEM((GRP, EK), jnp.int32)] * 2,
        [pltpu.VMEM((GRP, EK), jnp.int32)] * 2,
        [pltpu.VMEM((EK, H), jnp.float32)] * NSLOT,
        pltpu.VMEM_SHARED((NP, H), jnp.float32),
        [pltpu.SemaphoreType.DMA] * NSLOT,
        [pltpu.SemaphoreType.DMA] * 2,
    ],
)
def _sc_agg(h_hbm, src_hbm, dst_hbm, out_hbm, sidx, didx, rows, acc,
            gsem, isem):
    c = lax.axis_index("c")
    s = lax.axis_index("s")

    def fillz(k, _):
        rows[0][k // (H // LANES), pl.ds((k % (H // LANES)) * LANES, LANES)] = (
            jnp.zeros((LANES,), jnp.float32)
        )
        return 0

    lax.fori_loop(0, EK * (H // LANES), fillz, 0)

    def zero_acc(j, _):
        pltpu.sync_copy(rows[0], acc.at[pl.ds(s * ROWS_T + j * ZR, ZR)])
        return 0

    lax.fori_loop(0, ROWS_T // ZR, zero_acc, 0)
    wid = c * NS + s
    plsc.subcore_barrier()

    def load_idx_async(g, p):
        base = wid * NCHUNK + g * GRP
        pltpu.async_copy(src_hbm.at[pl.ds(base, GRP)], sidx[p], isem[p])
        pltpu.async_copy(dst_hbm.at[pl.ds(base, GRP)], didx[p], isem[p])

    def wait_idx(p):
        pltpu.make_async_copy(src_hbm.at[pl.ds(0, GRP)], sidx[p], isem[p]).wait()
        pltpu.make_async_copy(dst_hbm.at[pl.ds(0, GRP)], didx[p], isem[p]).wait()

    def fire_gather(idx_row, t):
        pltpu.async_copy(h_hbm.at[idx_row], rows[t], gsem[t])

    def process_group(p, cross):
        for j in range(GRP):
            t = j % NSLOT
            if j < GRP - 1:
                fire_gather(sidx[p].at[j + 1], (j + 1) % NSLOT)
            elif cross:
                fire_gather(sidx[1 - p].at[0], 0)
            pltpu.make_async_copy(h_hbm.at[sidx[p].at[j]], rows[t], gsem[t]).wait()
            pltpu.sync_copy(rows[t], acc.at[didx[p].at[j]], add=True)

    pltpu.sync_copy(src_hbm.at[pl.ds(wid * NCHUNK, GRP)], sidx[0])
    pltpu.sync_copy(dst_hbm.at[pl.ds(wid * NCHUNK, GRP)], didx[0])
    load_idx_async(1, 1)
    fire_gather(sidx[0].at[0], 0)

    def super_body(sg, _):
        ga = 2 * sg

        @pl.when(sg > 0)
        def _():
            wait_idx(0)
            fire_gather(sidx[0].at[0], 0)

        wait_idx(1)
        process_group(0, cross=True)

        @pl.when(sg < NGRP // 2 - 1)
        def _():
            load_idx_async(ga + 2, 0)

        process_group(1, cross=False)

        @pl.when(sg < NGRP // 2 - 1)
        def _():
            load_idx_async(ga + 3, 1)

        return 0

    lax.fori_loop(0, NGRP // 2, super_body, 0)
    plsc.subcore_barrier()

    pltpu.sync_copy(
        acc.at[pl.ds(s * ROWS_T, ROWS_T)],
        out_hbm.at[c, pl.ds(s * ROWS_T, ROWS_T)],
    )


def _tc_mm1_body(x_ref, w_ref, degp_ref, h_ref, dinv_ref):
    dinv = lax.rsqrt(degp_ref[0, :N] + degp_ref[1, :N] + 1.0)
    h = jnp.dot(x_ref[...], w_ref[...], preferred_element_type=jnp.float32)
    dinv_ref[...] = dinv
    h_ref[...] = h * dinv


def _bn_relu(aggp, hp, dinv, b, g, be):
    z = (aggp[0, :N] + aggp[1, :N] + hp) * dinv + b
    mu = jnp.mean(z, axis=0, keepdims=True)
    var = jnp.mean((z - mu) ** 2, axis=0, keepdims=True)
    return jnp.maximum((z - mu) * lax.rsqrt(var + 1e-5) * g + be, 0.0)


def _tc_bn_mm_body(aggp_ref, hp_ref, dinv_ref, b_ref, g_ref, be_ref, w_ref,
                   o_ref):
    dinv = dinv_ref[...]
    y = _bn_relu(aggp_ref[...], hp_ref[...], dinv, b_ref[...], g_ref[...],
                 be_ref[...])
    o_ref[...] = jnp.dot(y, w_ref[...], preferred_element_type=jnp.float32) * dinv


def _tc_bn_pool_body(aggp_ref, hp_ref, dinv_ref, b_ref, g_ref, be_ref,
                     batch_ref, fc1w_ref, fc1b_ref, fc2w_ref, fc2b_ref, o_ref):
    dinv = dinv_ref[...]
    y = _bn_relu(aggp_ref[...], hp_ref[...], dinv, b_ref[...], g_ref[...],
                 be_ref[...])
    gid = lax.broadcasted_iota(jnp.int32, (G, N), 0)
    onehot_t = (batch_ref[...] == gid).astype(jnp.float32)
    sums = jnp.dot(onehot_t, y, preferred_element_type=jnp.float32)
    counts = jnp.sum(onehot_t, axis=1, keepdims=True)
    pooled = sums / jnp.maximum(counts, 1.0)
    a = jnp.maximum(
        jnp.dot(pooled, fc1w_ref[...], preferred_element_type=jnp.float32)
        + fc1b_ref[...],
        0.0,
    )
    o_ref[...] = (
        jnp.dot(a, fc2w_ref[...], preferred_element_type=jnp.float32)
        + fc2b_ref[...]
    )


def kernel(x, edge_index, batch, W1, b1, gamma1, beta1, W2, b2, gamma2, beta2,
           fc1_W, fc1_b, fc2_W, fc2_b):
    pad_iota = jnp.arange(EP - E, dtype=jnp.int32)
    src = jnp.concatenate([edge_index[0], pad_iota % N]).reshape(EP // EK, EK)
    dst = jnp.concatenate([edge_index[1], N + pad_iota % (NP - N)]).reshape(
        EP // EK, EK
    )

    degp = _sc_degree(dst)

    h1p, dinv = pl.pallas_call(
        _tc_mm1_body,
        out_shape=[
            jax.ShapeDtypeStruct((N, H), jnp.float32),
            jax.ShapeDtypeStruct((N, H), jnp.float32),
        ],
    )(x, W1, degp)

    agg1 = _sc_agg(h1p, src, dst)

    h2p = pl.pallas_call(
        _tc_bn_mm_body,
        out_shape=jax.ShapeDtypeStruct((N, H), jnp.float32),
    )(agg1, h1p, dinv, b1.reshape(1, H), gamma1.reshape(1, H),
      beta1.reshape(1, H), W2)

    agg2 = _sc_agg(h2p, src, dst)

    fc1w_p = jnp.pad(fc1_W, ((0, 0), (0, 128 - fc1_W.shape[1])))
    fc1b_p = jnp.pad(fc1_b, (0, 128 - fc1_b.shape[0])).reshape(1, 128)
    fc2w_p = jnp.pad(fc2_W, ((0, 128 - fc2_W.shape[0]), (0, 128 - fc2_W.shape[1])))
    fc2b_p = jnp.pad(fc2_b, (0, 128 - fc2_b.shape[0])).reshape(1, 128)

    out_p = pl.pallas_call(
        _tc_bn_pool_body,
        out_shape=jax.ShapeDtypeStruct((G, 128), jnp.float32),
    )(agg2, h2p, dinv, b2.reshape(1, H), gamma2.reshape(1, H),
      beta2.reshape(1, H), batch.reshape(1, N), fc1w_p, fc1b_p, fc2w_p, fc2b_p)

    return out_p[:, : fc2_W.shape[1]]
